# baseline (speedup 1.0000x reference)
import jax
import jax.numpy as jnp
from jax.experimental import pallas as pl
from jax.experimental.pallas import tpu as pltpu

_ROWS = 10000
_FEAT = 128
_CHUNK = 1250
_NCHUNK = _ROWS // _CHUNK


def _linear_kernel(x_hbm, w_ref, b_ref, o_hbm, xbuf, obuf, sin, sout):
    wb = w_ref[...].astype(jnp.bfloat16)
    bb = b_ref[...]

    def in_copy(i, slot):
        return pltpu.make_async_copy(
            x_hbm.at[pl.ds(i * _CHUNK, _CHUNK), :], xbuf.at[slot], sin.at[slot])

    def out_copy(i, slot):
        return pltpu.make_async_copy(
            obuf.at[slot], o_hbm.at[pl.ds(i * _CHUNK, _CHUNK), :], sout.at[slot])

    in_copy(0, 0).start()
    for i in range(_NCHUNK):
        slot = i % 2
        if i + 1 < _NCHUNK:
            in_copy(i + 1, (i + 1) % 2).start()
        in_copy(i, slot).wait()
        if i >= 2:
            out_copy(i - 2, slot).wait()
        acc = jax.lax.dot_general(
            xbuf[slot].astype(jnp.bfloat16), wb,
            dimension_numbers=(((1,), (1,)), ((), ())),
            preferred_element_type=jnp.float32,
        )
        obuf[slot] = acc + bb
        out_copy(i, slot).start()
    out_copy(_NCHUNK - 2, (_NCHUNK - 2) % 2).wait()
    out_copy(_NCHUNK - 1, (_NCHUNK - 1) % 2).wait()


def kernel(edge_index, x, W, b):
    del edge_index
    b2 = b.reshape(1, _FEAT)
    out = pl.pallas_call(
        _linear_kernel,
        in_specs=[
            pl.BlockSpec(memory_space=pl.ANY),
            pl.BlockSpec(memory_space=pltpu.MemorySpace.VMEM),
            pl.BlockSpec(memory_space=pltpu.MemorySpace.VMEM),
        ],
        out_specs=pl.BlockSpec(memory_space=pl.ANY),
        out_shape=jax.ShapeDtypeStruct((_ROWS, _FEAT), jnp.float32),
        scratch_shapes=[
            pltpu.VMEM((2, _CHUNK, _FEAT), jnp.float32),
            pltpu.VMEM((2, _CHUNK, _FEAT), jnp.float32),
            pltpu.SemaphoreType.DMA((2,)),
            pltpu.SemaphoreType.DMA((2,)),
        ],
    )(x, W, b2)
    return out
